# Initial kernel scaffold; baseline (speedup 1.0000x reference)
import functools

import jax
import jax.numpy as jnp
from jax import lax
from jax.experimental import pallas as pl
from jax.experimental.pallas import tpu as pltpu

NEG_POS_RATIO = 3


def _loss_kernel(cls_ref, locp_ref, loct_ref, tgt_ref, out_ref,
                 bg_scr, facc, iacc, *, nb, bn, n, c):
    b = pl.program_id(0)
    j = pl.program_id(1)

    x = cls_ref[0]
    tgt = tgt_ref[0, 0, :]
    pos = tgt != 0

    m = jnp.max(x, axis=1)
    e = jnp.exp(x - m[:, None])
    lse = m + jnp.log(jnp.sum(e, axis=1))
    cls_iota = lax.broadcasted_iota(jnp.int32, (bn, c), 1)
    x_tgt = jnp.sum(jnp.where(cls_iota == tgt[:, None], x, 0.0), axis=1)
    nll_tgt = lse - x_tgt
    bg = lse - x[:, 0]
    bg_scr[j, :] = jnp.where(pos, -1.0, bg)

    d = locp_ref[0] - loct_ref[0]
    ad = jnp.abs(d)
    sl1 = jnp.where(ad < 1.0, 0.5 * d * d, ad - 0.5)

    loc_c = jnp.sum(jnp.where(pos[:, None], sl1, 0.0))
    clsp_c = jnp.sum(jnp.where(pos, nll_tgt, 0.0))
    np_c = jnp.sum(pos.astype(jnp.int32))

    @pl.when((b == 0) & (j == 0))
    def _():
        facc[2] = 0.0

    @pl.when(j == 0)
    def _():
        facc[0] = loc_c
        facc[1] = clsp_c
        iacc[0] = np_c

    @pl.when(j > 0)
    def _():
        facc[0] += loc_c
        facc[1] += clsp_c
        iacc[0] += np_c

    @pl.when(j == nb - 1)
    def _():
        num_pos = iacc[0]
        loc_sum = facc[0]
        cls_pos = facc[1]
        k = jnp.minimum(NEG_POS_RATIO * num_pos, n - num_pos)

        bgv = bg_scr[:, :]
        bits = lax.bitcast_convert_type(bgv, jnp.int32)

        def search(i, t):
            cand = t | lax.shift_left(jnp.int32(1), 30 - i)
            cnt = jnp.sum((bits >= cand).astype(jnp.int32))
            return jnp.where(cnt >= k, cand, t)

        t = lax.fori_loop(0, 31, search, jnp.int32(0))
        kth = lax.bitcast_convert_type(t, jnp.float32)
        gt = bgv > kth
        sum_gt = jnp.sum(jnp.where(gt, bgv, 0.0))
        cnt_gt = jnp.sum(gt.astype(jnp.int32))
        cls_neg = sum_gt + (k - cnt_gt).astype(jnp.float32) * kth
        cls_neg = jnp.where(k > 0, cls_neg, 0.0)

        denom = jnp.maximum(num_pos, 1).astype(jnp.float32)
        batch_loss = jnp.where(
            num_pos > 0, (loc_sum + cls_pos + cls_neg) / denom, 0.0)
        facc[2] += batch_loss

    @pl.when((b == pl.num_programs(0) - 1) & (j == nb - 1))
    def _():
        out_ref[0, 0] = facc[2]


@jax.jit
def kernel(loc_preds, cls_preds, loc_targets, cls_targets):
    b, n, c = cls_preds.shape
    bn = 2048 if n % 2048 == 0 else n
    nb = n // bn

    tgt3 = cls_targets.reshape(b * nb, 1, bn)

    out = pl.pallas_call(
        functools.partial(_loss_kernel, nb=nb, bn=bn, n=n, c=c),
        grid=(b, nb),
        in_specs=[
            pl.BlockSpec((1, bn, c), lambda bi, ji: (bi, ji, 0)),
            pl.BlockSpec((1, bn, 4), lambda bi, ji: (bi, ji, 0)),
            pl.BlockSpec((1, bn, 4), lambda bi, ji: (bi, ji, 0)),
            pl.BlockSpec((1, 1, bn), lambda bi, ji: (bi * nb + ji, 0, 0)),
        ],
        out_specs=pl.BlockSpec((1, 1), lambda bi, ji: (0, 0)),
        out_shape=jax.ShapeDtypeStruct((1, 1), jnp.float32),
        scratch_shapes=[
            pltpu.VMEM((nb, bn), jnp.float32),
            pltpu.SMEM((3,), jnp.float32),
            pltpu.SMEM((1,), jnp.int32),
        ],
    )(cls_preds, loc_preds, loc_targets, tgt3)
    return out.reshape(())


# speedup vs baseline: 1.1380x; 1.1380x over previous
import functools

import jax
import jax.numpy as jnp
from jax import lax
from jax.experimental import pallas as pl
from jax.experimental.pallas import tpu as pltpu

NEG_POS_RATIO = 3


def _loss_kernel(cls_ref, locp_ref, loct_ref, tgt_ref, out_ref,
                 bg_scr, facc, iacc, *, nb, bn, n, c):
    b = pl.program_id(0)
    j = pl.program_id(1)

    x = cls_ref[0]
    tgt = tgt_ref[0, 0, :]
    pos = tgt != 0

    m = jnp.max(x, axis=1)
    e = jnp.exp(x - m[:, None])
    lse = m + jnp.log(jnp.sum(e, axis=1))
    cls_iota = lax.broadcasted_iota(jnp.int32, (bn, c), 1)
    x_tgt = jnp.sum(jnp.where(cls_iota == tgt[:, None], x, 0.0), axis=1)
    nll_tgt = lse - x_tgt
    bg = lse - x[:, 0]
    bg_scr[j, :] = jnp.where(pos, -1.0, bg)

    d = locp_ref[0] - loct_ref[0]
    ad = jnp.abs(d)
    sl1 = jnp.where(ad < 1.0, 0.5 * d * d, ad - 0.5)

    sl1_row = jnp.sum(sl1, axis=1)
    loc_c = jnp.sum(jnp.where(pos, sl1_row, 0.0))
    clsp_c = jnp.sum(jnp.where(pos, nll_tgt, 0.0))
    np_c = jnp.sum(pos.astype(jnp.int32))

    @pl.when((b == 0) & (j == 0))
    def _():
        facc[2] = 0.0

    @pl.when(j == 0)
    def _():
        facc[0] = loc_c
        facc[1] = clsp_c
        iacc[0] = np_c

    @pl.when(j > 0)
    def _():
        facc[0] += loc_c
        facc[1] += clsp_c
        iacc[0] += np_c

    @pl.when(j == nb - 1)
    def _():
        num_pos = iacc[0]
        loc_sum = facc[0]
        cls_pos = facc[1]
        k = jnp.minimum(NEG_POS_RATIO * num_pos, n - num_pos)

        bgv = bg_scr[:, :]
        bits = lax.bitcast_convert_type(bgv, jnp.int32)

        def search(i, t):
            cand = t | lax.shift_left(jnp.int32(1), 30 - i)
            cnt = jnp.sum((bits >= cand).astype(jnp.int32))
            return jnp.where(cnt >= k, cand, t)

        t = lax.fori_loop(0, 31, search, jnp.int32(0))
        kth = lax.bitcast_convert_type(t, jnp.float32)
        gt = bgv > kth
        sum_gt = jnp.sum(jnp.where(gt, bgv, 0.0))
        cnt_gt = jnp.sum(gt.astype(jnp.int32))
        cls_neg = sum_gt + (k - cnt_gt).astype(jnp.float32) * kth
        cls_neg = jnp.where(k > 0, cls_neg, 0.0)

        denom = jnp.maximum(num_pos, 1).astype(jnp.float32)
        batch_loss = jnp.where(
            num_pos > 0, (loc_sum + cls_pos + cls_neg) / denom, 0.0)
        facc[2] += batch_loss

    @pl.when((b == pl.num_programs(0) - 1) & (j == nb - 1))
    def _():
        out_ref[0, 0] = facc[2]


@jax.jit
def kernel(loc_preds, cls_preds, loc_targets, cls_targets):
    b, n, c = cls_preds.shape
    bn = 2048 if n % 2048 == 0 else n
    nb = n // bn

    tgt3 = cls_targets.reshape(b * nb, 1, bn)

    out = pl.pallas_call(
        functools.partial(_loss_kernel, nb=nb, bn=bn, n=n, c=c),
        grid=(b, nb),
        in_specs=[
            pl.BlockSpec((1, bn, c), lambda bi, ji: (bi, ji, 0)),
            pl.BlockSpec((1, bn, 4), lambda bi, ji: (bi, ji, 0)),
            pl.BlockSpec((1, bn, 4), lambda bi, ji: (bi, ji, 0)),
            pl.BlockSpec((1, 1, bn), lambda bi, ji: (bi * nb + ji, 0, 0)),
        ],
        out_specs=pl.BlockSpec(memory_space=pltpu.SMEM),
        out_shape=jax.ShapeDtypeStruct((1, 1), jnp.float32),
        scratch_shapes=[
            pltpu.VMEM((nb, bn), jnp.float32),
            pltpu.SMEM((3,), jnp.float32),
            pltpu.SMEM((1,), jnp.int32),
        ],
    )(cls_preds, loc_preds, loc_targets, tgt3)
    return out.reshape(())


# speedup vs baseline: 1.7200x; 1.5114x over previous
import functools

import jax
import jax.numpy as jnp
from jax import lax
from jax.experimental import pallas as pl
from jax.experimental.pallas import tpu as pltpu
from jax.experimental.pallas import tpu_sc as plsc

NEG_POS_RATIO = 3


def _cls_kernel(cls_ref, tgt_ref, out_ref, bg_scr, facc, iacc, *, nb, bn, n, c):
    b = pl.program_id(0)
    j = pl.program_id(1)

    x = cls_ref[0]
    tgt = tgt_ref[0, 0, :]
    pos = tgt != 0

    m = jnp.max(x, axis=1)
    e = jnp.exp(x - m[:, None])
    lse = m + jnp.log(jnp.sum(e, axis=1))
    cls_iota = lax.broadcasted_iota(jnp.int32, (bn, c), 1)
    sel = (cls_iota == tgt[:, None]) & (cls_iota != 0)
    x_tgt_sum = jnp.sum(jnp.where(sel, x, 0.0))
    lse_pos = jnp.sum(jnp.where(pos, lse, 0.0))
    clsp_c = lse_pos - x_tgt_sum
    bg = lse - x[:, 0]
    bg_scr[j, :] = jnp.where(pos, -1.0, bg)
    np_c = jnp.sum(pos.astype(jnp.int32))

    @pl.when((b == 0) & (j == 0))
    def _():
        facc[1] = 0.0

    @pl.when(j == 0)
    def _():
        facc[0] = clsp_c
        iacc[0] = np_c

    @pl.when(j > 0)
    def _():
        facc[0] += clsp_c
        iacc[0] += np_c

    @pl.when(j == nb - 1)
    def _():
        num_pos = iacc[0]
        cls_pos = facc[0]
        k = jnp.minimum(NEG_POS_RATIO * num_pos, n - num_pos)

        bgv = bg_scr[:, :]
        bits = lax.bitcast_convert_type(bgv, jnp.int32)

        def search(i, t):
            cand = t | lax.shift_left(jnp.int32(1), 30 - i)
            cnt = jnp.sum((bits >= cand).astype(jnp.int32))
            return jnp.where(cnt >= k, cand, t)

        t = lax.fori_loop(0, 31, search, jnp.int32(0))
        kth = lax.bitcast_convert_type(t, jnp.float32)
        gt = bgv > kth
        sum_gt = jnp.sum(jnp.where(gt, bgv, 0.0))
        cnt_gt = jnp.sum(gt.astype(jnp.int32))
        cls_neg = sum_gt + (k - cnt_gt).astype(jnp.float32) * kth
        cls_neg = jnp.where(k > 0, cls_neg, 0.0)

        denom = jnp.maximum(num_pos, 1).astype(jnp.float32)
        facc[1] += jnp.where(num_pos > 0, (cls_pos + cls_neg) / denom, 0.0)

    @pl.when((b == pl.num_programs(0) - 1) & (j == nb - 1))
    def _():
        out_ref[0, 0] = facc[1]


def _tc_cls_loss(cls_preds, cls_targets):
    b, n, c = cls_preds.shape
    bn = 2048 if n % 2048 == 0 else n
    nb = n // bn
    tgt3 = cls_targets.reshape(b * nb, 1, bn)
    out = pl.pallas_call(
        functools.partial(_cls_kernel, nb=nb, bn=bn, n=n, c=c),
        grid=(b, nb),
        in_specs=[
            pl.BlockSpec((1, bn, c), lambda bi, ji: (bi, ji, 0)),
            pl.BlockSpec((1, 1, bn), lambda bi, ji: (bi * nb + ji, 0, 0)),
        ],
        out_specs=pl.BlockSpec(memory_space=pltpu.SMEM),
        out_shape=jax.ShapeDtypeStruct((1, 1), jnp.float32),
        scratch_shapes=[
            pltpu.VMEM((nb, bn), jnp.float32),
            pltpu.SMEM((2,), jnp.float32),
            pltpu.SMEM((1,), jnp.int32),
        ],
    )(cls_preds, tgt3)
    return out.reshape(())


def _make_sc_loc(b, n):
    info = plsc.get_sparse_core_info()
    nc, ns = info.num_cores, info.num_subcores
    nw = nc * ns
    assert nw == 2 * b
    n2 = n // 2
    nv = n2 * 4
    mesh = plsc.VectorSubcoreMesh(core_axis_name="c", subcore_axis_name="s")

    @functools.partial(
        pl.kernel, mesh=mesh,
        out_type=jax.ShapeDtypeStruct((nw, 2, 16), jnp.float32),
        scratch_types=[
            pltpu.VMEM((4, n2), jnp.float32),
            pltpu.VMEM((4, n2), jnp.float32),
            pltpu.VMEM((n2,), jnp.int32),
            pltpu.VMEM((2, 16), jnp.float32),
        ],
    )
    def sc_loc(locp_hbm, loct_hbm, tgt_hbm, out_hbm, lp_v, lt_v, tg_v, o_v):
        wid = lax.axis_index("s") * nc + lax.axis_index("c")
        batch = wid // 2
        half = wid % 2
        off = batch * n + half * n2
        for cc in range(4):
            pltpu.sync_copy(locp_hbm.at[pl.ds(cc * (b * n) + off, n2)],
                            lp_v.at[cc])
            pltpu.sync_copy(loct_hbm.at[pl.ds(cc * (b * n) + off, n2)],
                            lt_v.at[cc])
        pltpu.sync_copy(tgt_hbm.at[pl.ds(off, n2)], tg_v)

        def body(g, carry):
            acc, cnt = carry
            tv = tg_v[pl.ds(g * 16, 16)]
            m = tv != 0
            cnt = cnt + jnp.where(m, 1, 0)
            for cc in range(4):
                d = lp_v[cc, pl.ds(g * 16, 16)] - lt_v[cc, pl.ds(g * 16, 16)]
                ad = jnp.abs(d)
                sl1 = jnp.where(ad < 1.0, 0.5 * d * d, ad - 0.5)
                acc = acc + jnp.where(m, sl1, 0.0)
            return acc, cnt

        acc, cnt = lax.fori_loop(
            0, n2 // 16, body,
            (jnp.zeros((16,), jnp.float32), jnp.zeros((16,), jnp.int32)))
        o_v[0, :] = acc
        o_v[1, :] = cnt.astype(jnp.float32)
        pltpu.sync_copy(o_v, out_hbm.at[wid])

    return sc_loc


@jax.jit
def kernel(loc_preds, cls_preds, loc_targets, cls_targets):
    b, n, _ = cls_preds.shape

    sc_loc = _make_sc_loc(b, n)
    sc_out = sc_loc(jnp.transpose(loc_preds, (2, 0, 1)).reshape(-1),
                    jnp.transpose(loc_targets, (2, 0, 1)).reshape(-1),
                    cls_targets.reshape(-1))

    cls_loss = _tc_cls_loss(cls_preds, cls_targets)

    loc_b = sc_out[:, 0, :].reshape(b, 2, 16).sum(axis=(1, 2))
    np_b = sc_out[:, 1, :].reshape(b, 2, 16).sum(axis=(1, 2))
    loc_loss = jnp.sum(
        jnp.where(np_b > 0, loc_b / jnp.maximum(np_b, 1.0), 0.0))
    return cls_loss + loc_loss


# speedup vs baseline: 3.3578x; 1.9522x over previous
import functools

import jax
import jax.numpy as jnp
from jax import lax
from jax.experimental import pallas as pl
from jax.experimental.pallas import tpu as pltpu
from jax.experimental.pallas import tpu_sc as plsc

NEG_POS_RATIO = 3


def _cls_kernel(cls_ref, tgt_ref, out_ref, bg_scr, facc, iacc, *, nb, bn, n, c):
    b = pl.program_id(0)
    j = pl.program_id(1)

    x = cls_ref[0]
    tgt = tgt_ref[0, 0, :]
    pos = tgt != 0

    m = jnp.max(x, axis=0)
    e = jnp.exp(x - m[None, :])
    lse = m + jnp.log(jnp.sum(e, axis=0))
    cls_iota = lax.broadcasted_iota(jnp.int32, (c, bn), 0)
    sel = (cls_iota == tgt[None, :]) & (cls_iota != 0)
    x_tgt_sum = jnp.sum(jnp.where(sel, x, 0.0))
    lse_pos = jnp.sum(jnp.where(pos, lse, 0.0))
    clsp_c = lse_pos - x_tgt_sum
    bg = lse - x[0, :]
    bg_scr[j, :] = jnp.where(pos, -1.0, bg)
    np_c = jnp.sum(pos.astype(jnp.int32))

    @pl.when((b == 0) & (j == 0))
    def _():
        facc[1] = 0.0

    @pl.when(j == 0)
    def _():
        facc[0] = clsp_c
        iacc[0] = np_c

    @pl.when(j > 0)
    def _():
        facc[0] += clsp_c
        iacc[0] += np_c

    @pl.when(j == nb - 1)
    def _():
        num_pos = iacc[0]
        cls_pos = facc[0]
        k = jnp.minimum(NEG_POS_RATIO * num_pos, n - num_pos)

        bgv = bg_scr[:, :]
        bits = lax.bitcast_convert_type(bgv, jnp.int32)

        def search(i, t):
            cand = t | lax.shift_left(jnp.int32(1), 30 - i)
            cnt = jnp.sum((bits >= cand).astype(jnp.int32))
            return jnp.where(cnt >= k, cand, t)

        t = lax.fori_loop(0, 31, search, jnp.int32(0))
        kth = lax.bitcast_convert_type(t, jnp.float32)
        gt = bgv > kth
        sum_gt = jnp.sum(jnp.where(gt, bgv, 0.0))
        cnt_gt = jnp.sum(gt.astype(jnp.int32))
        cls_neg = sum_gt + (k - cnt_gt).astype(jnp.float32) * kth
        cls_neg = jnp.where(k > 0, cls_neg, 0.0)

        denom = jnp.maximum(num_pos, 1).astype(jnp.float32)
        facc[1] += jnp.where(num_pos > 0, (cls_pos + cls_neg) / denom, 0.0)

    @pl.when((b == pl.num_programs(0) - 1) & (j == nb - 1))
    def _():
        out_ref[0, 0] = facc[1]


def _tc_cls_loss(cls_t, cls_targets):
    b, c, n = cls_t.shape
    bn = 2048 if n % 2048 == 0 else n
    nb = n // bn
    tgt3 = cls_targets.reshape(b * nb, 1, bn)
    out = pl.pallas_call(
        functools.partial(_cls_kernel, nb=nb, bn=bn, n=n, c=c),
        grid=(b, nb),
        in_specs=[
            pl.BlockSpec((1, c, bn), lambda bi, ji: (bi, 0, ji)),
            pl.BlockSpec((1, 1, bn), lambda bi, ji: (bi * nb + ji, 0, 0)),
        ],
        out_specs=pl.BlockSpec(memory_space=pltpu.SMEM),
        out_shape=jax.ShapeDtypeStruct((1, 1), jnp.float32),
        scratch_shapes=[
            pltpu.VMEM((nb, bn), jnp.float32),
            pltpu.SMEM((2,), jnp.float32),
            pltpu.SMEM((1,), jnp.int32),
        ],
    )(cls_t, tgt3)
    return out.reshape(())


def _make_sc_loc(b, n):
    info = plsc.get_sparse_core_info()
    nc, ns = info.num_cores, info.num_subcores
    nw = nc * ns
    assert nw == 2 * b
    n2 = n // 2
    nv = n2 * 4
    mesh = plsc.VectorSubcoreMesh(core_axis_name="c", subcore_axis_name="s")

    @functools.partial(
        pl.kernel, mesh=mesh,
        out_type=jax.ShapeDtypeStruct((nw, 2, 16), jnp.float32),
        scratch_types=[
            pltpu.VMEM((4, n2), jnp.float32),
            pltpu.VMEM((4, n2), jnp.float32),
            pltpu.VMEM((n2,), jnp.int32),
            pltpu.VMEM((2, 16), jnp.float32),
        ],
    )
    def sc_loc(locp_hbm, loct_hbm, tgt_hbm, out_hbm, lp_v, lt_v, tg_v, o_v):
        wid = lax.axis_index("s") * nc + lax.axis_index("c")
        batch = wid // 2
        half = wid % 2
        off = batch * n + half * n2
        for cc in range(4):
            pltpu.sync_copy(locp_hbm.at[pl.ds(cc * (b * n) + off, n2)],
                            lp_v.at[cc])
            pltpu.sync_copy(loct_hbm.at[pl.ds(cc * (b * n) + off, n2)],
                            lt_v.at[cc])
        pltpu.sync_copy(tgt_hbm.at[pl.ds(off, n2)], tg_v)

        def body(g, carry):
            acc, cnt = carry
            tv = tg_v[pl.ds(g * 16, 16)]
            m = tv != 0
            cnt = cnt + jnp.where(m, 1, 0)
            for cc in range(4):
                d = lp_v[cc, pl.ds(g * 16, 16)] - lt_v[cc, pl.ds(g * 16, 16)]
                ad = jnp.abs(d)
                sl1 = jnp.where(ad < 1.0, 0.5 * d * d, ad - 0.5)
                acc = acc + jnp.where(m, sl1, 0.0)
            return acc, cnt

        acc, cnt = lax.fori_loop(
            0, n2 // 16, body,
            (jnp.zeros((16,), jnp.float32), jnp.zeros((16,), jnp.int32)))
        o_v[0, :] = acc
        o_v[1, :] = cnt.astype(jnp.float32)
        pltpu.sync_copy(o_v, out_hbm.at[wid])

    return sc_loc


@jax.jit
def kernel(loc_preds, cls_preds, loc_targets, cls_targets):
    b, n, _ = cls_preds.shape

    sc_loc = _make_sc_loc(b, n)
    sc_out = sc_loc(jnp.transpose(loc_preds, (2, 0, 1)).reshape(-1),
                    jnp.transpose(loc_targets, (2, 0, 1)).reshape(-1),
                    cls_targets.reshape(-1))

    cls_loss = _tc_cls_loss(jnp.transpose(cls_preds, (0, 2, 1)), cls_targets)

    loc_b = sc_out[:, 0, :].reshape(b, 2, 16).sum(axis=(1, 2))
    np_b = sc_out[:, 1, :].reshape(b, 2, 16).sum(axis=(1, 2))
    loc_loss = jnp.sum(
        jnp.where(np_b > 0, loc_b / jnp.maximum(np_b, 1.0), 0.0))
    return cls_loss + loc_loss
